# baseline (speedup 1.0000x reference)
import functools

import jax
import jax.numpy as jnp
from jax import lax
from jax.experimental import pallas as pl
from jax.experimental.pallas import tpu as pltpu
from jax.experimental.pallas import tpu_sc as plsc

D_MODEL = 64
SCALE = float(D_MODEL) ** 0.5

_NC = 2
_NS = 16
_NW = _NC * _NS

_IDX_W = 128
_CHUNK_ROWS = 512
_GPC = _CHUNK_ROWS // _IDX_W


@functools.cache
def _make_lookup(B, V, D):
    assert B % (_NW * _CHUNK_ROWS) == 0
    assert D % 16 == 0
    b_per_w = B // _NW
    n_chunks = b_per_w // _CHUNK_ROWS
    idx_rows_per_w = b_per_w // _IDX_W
    mesh = plsc.VectorSubcoreMesh(core_axis_name="c", subcore_axis_name="s")

    @functools.partial(
        pl.kernel,
        out_type=jax.ShapeDtypeStruct((B, D), jnp.float32),
        mesh=mesh,
        scratch_types=[
            pltpu.VMEM((_GPC, _IDX_W), jnp.int32),
            pltpu.VMEM((_CHUNK_ROWS, D), jnp.float32),
            pltpu.SemaphoreType.DMA,
        ],
        compiler_params=pltpu.CompilerParams(use_tc_tiling_on_sc=False),
    )
    def lookup(idx_hbm, table_hbm, out_hbm, idx_v, rows_v, gsem):
        wid = lax.axis_index("s") * _NC + lax.axis_index("c")
        idx_row0 = wid * idx_rows_per_w
        out_row0 = wid * b_per_w

        @pl.loop(0, n_chunks)
        def chunk_loop(i):
            pltpu.sync_copy(idx_hbm.at[pl.ds(idx_row0 + i * _GPC, _GPC)], idx_v)
            copies = [
                pltpu.async_copy(
                    table_hbm.at[idx_v.at[j]],
                    rows_v.at[pl.ds(j * _IDX_W, _IDX_W)],
                    gsem,
                )
                for j in range(_GPC)
            ]
            for c in copies:
                c.wait()

            @pl.loop(0, _CHUNK_ROWS)
            def scale_loop(r):
                for l in range(D // 16):
                    rows_v[r, pl.ds(l * 16, 16)] = (
                        rows_v[r, pl.ds(l * 16, 16)] * SCALE
                    )

            pltpu.sync_copy(
                rows_v, out_hbm.at[pl.ds(out_row0 + i * _CHUNK_ROWS, _CHUNK_ROWS)]
            )

    return lookup


def kernel(x, table):
    B = x.shape[0] * x.shape[1]
    V, D = table.shape
    idx = x.reshape(B // _IDX_W, _IDX_W).astype(jnp.int32)
    out = _make_lookup(B, V, D)(idx, table)
    return out.reshape(x.shape[0], x.shape[1], D)


# speedup vs baseline: 1.1185x; 1.1185x over previous
import functools

import jax
import jax.numpy as jnp
from jax import lax
from jax.experimental import pallas as pl
from jax.experimental.pallas import tpu as pltpu
from jax.experimental.pallas import tpu_sc as plsc

D_MODEL = 64
SCALE = float(D_MODEL) ** 0.5

_NC = 2
_NS = 16
_NW = _NC * _NS

_IDX_W = 128
_CHUNK_ROWS = 512
_GPC = _CHUNK_ROWS // _IDX_W


@functools.cache
def _make_lookup(B, V, D):
    assert B % (_NW * _CHUNK_ROWS) == 0
    assert D % 16 == 0
    b_per_w = B // _NW
    n_chunks = b_per_w // _CHUNK_ROWS
    assert n_chunks >= 2
    idx_rows_per_w = b_per_w // _IDX_W
    mesh = plsc.VectorSubcoreMesh(core_axis_name="c", subcore_axis_name="s")

    @functools.partial(
        pl.kernel,
        out_type=jax.ShapeDtypeStruct((B, D), jnp.float32),
        mesh=mesh,
        scratch_types=[
            pltpu.VMEM((_GPC, _IDX_W), jnp.int32),
            pltpu.VMEM((_GPC, _IDX_W), jnp.int32),
            pltpu.VMEM((_CHUNK_ROWS, D), jnp.float32),
            pltpu.VMEM((_CHUNK_ROWS, D), jnp.float32),
            pltpu.SemaphoreType.DMA,
            pltpu.SemaphoreType.DMA,
            pltpu.SemaphoreType.DMA,
            pltpu.SemaphoreType.DMA,
        ],
        compiler_params=pltpu.CompilerParams(use_tc_tiling_on_sc=False),
    )
    def lookup(idx_hbm, table_hbm, out_hbm, idx0, idx1, rows0, rows1,
               gsem0, gsem1, wsem0, wsem1):
        wid = lax.axis_index("s") * _NC + lax.axis_index("c")
        idx_row0 = wid * idx_rows_per_w
        out_row0 = wid * b_per_w
        idx_v = (idx0, idx1)
        rows_v = (rows0, rows1)
        gsem = (gsem0, gsem1)
        wsem = (wsem0, wsem1)

        def stage_and_fire(c, b):
            pltpu.sync_copy(idx_hbm.at[pl.ds(idx_row0 + c * _GPC, _GPC)],
                            idx_v[b])
            for j in range(_GPC):
                pltpu.async_copy(
                    table_hbm.at[idx_v[b].at[j]],
                    rows_v[b].at[pl.ds(j * _IDX_W, _IDX_W)],
                    gsem[b],
                )

        def drain_gathers(b):
            for j in range(_GPC):
                pltpu.make_async_copy(
                    table_hbm.at[idx_v[b].at[j]],
                    rows_v[b].at[pl.ds(j * _IDX_W, _IDX_W)],
                    gsem[b],
                ).wait()

        def scale_buf(b):
            @pl.loop(0, _CHUNK_ROWS, step=4)
            def scale_loop(r):
                for dr in range(4):
                    for l in range(D // 16):
                        rows_v[b][r + dr, pl.ds(l * 16, 16)] = (
                            rows_v[b][r + dr, pl.ds(l * 16, 16)] * SCALE
                        )

        def fire_writeback(c, b):
            pltpu.async_copy(
                rows_v[b],
                out_hbm.at[pl.ds(out_row0 + c * _CHUNK_ROWS, _CHUNK_ROWS)],
                wsem[b],
            )

        def wait_writeback(c, b):
            pltpu.make_async_copy(
                rows_v[b],
                out_hbm.at[pl.ds(out_row0 + c * _CHUNK_ROWS, _CHUNK_ROWS)],
                wsem[b],
            ).wait()

        stage_and_fire(0, 0)
        stage_and_fire(1, 1)

        @pl.loop(0, n_chunks - 2)
        def chunk_loop(c):
            b = lax.rem(c, 2)

            @pl.when(b == 0)
            def _():
                drain_gathers(0)
                scale_buf(0)
                fire_writeback(c, 0)
                wait_writeback(c, 0)
                stage_and_fire(c + 2, 0)

            @pl.when(b == 1)
            def _():
                drain_gathers(1)
                scale_buf(1)
                fire_writeback(c, 1)
                wait_writeback(c, 1)
                stage_and_fire(c + 2, 1)

        for c, b in ((n_chunks - 2, (n_chunks - 2) % 2),
                     (n_chunks - 1, (n_chunks - 1) % 2)):
            drain_gathers(b)
            scale_buf(b)
            fire_writeback(c, b)
            wait_writeback(c, b)

    return lookup


def kernel(x, table):
    B = x.shape[0] * x.shape[1]
    V, D = table.shape
    idx = x.reshape(B // _IDX_W, _IDX_W).astype(jnp.int32)
    out = _make_lookup(B, V, D)(idx, table)
    return out.reshape(x.shape[0], x.shape[1], D)


# speedup vs baseline: 1.1288x; 1.0092x over previous
import functools

import jax
import jax.numpy as jnp
from jax import lax
from jax.experimental import pallas as pl
from jax.experimental.pallas import tpu as pltpu
from jax.experimental.pallas import tpu_sc as plsc

_NC = 2
_NS = 16
_NW = _NC * _NS

_XR_PER_CHUNK = 4


@functools.cache
def _make_lookup(N, T, V, D):
    scale = float(D) ** 0.5
    assert N % (_NW * _XR_PER_CHUNK) == 0
    assert D % 16 == 0
    xr_per_w = N // _NW
    n_chunks = xr_per_w // _XR_PER_CHUNK
    assert n_chunks >= 2 and n_chunks % 2 == 0
    mesh = plsc.VectorSubcoreMesh(core_axis_name="c", subcore_axis_name="s")

    @functools.partial(
        pl.kernel,
        out_type=jax.ShapeDtypeStruct((N, T, D), jnp.float32),
        mesh=mesh,
        scratch_types=[
            pltpu.VMEM((_XR_PER_CHUNK, T), jnp.int32),
            pltpu.VMEM((_XR_PER_CHUNK, T), jnp.int32),
            pltpu.VMEM((_XR_PER_CHUNK, T, D), jnp.float32),
            pltpu.VMEM((_XR_PER_CHUNK, T, D), jnp.float32),
            pltpu.SemaphoreType.DMA,
            pltpu.SemaphoreType.DMA,
            pltpu.SemaphoreType.DMA,
            pltpu.SemaphoreType.DMA,
        ],
        compiler_params=pltpu.CompilerParams(use_tc_tiling_on_sc=False),
    )
    def lookup(idx_hbm, table_hbm, out_hbm, idx0, idx1, rows0, rows1,
               gsem0, gsem1, wsem0, wsem1):
        wid = lax.axis_index("s") * _NC + lax.axis_index("c")
        row0 = wid * xr_per_w
        idx_v = (idx0, idx1)
        rows_v = (rows0, rows1)
        gsem = (gsem0, gsem1)
        wsem = (wsem0, wsem1)

        def stage_and_fire(c, b):
            pltpu.sync_copy(
                idx_hbm.at[pl.ds(row0 + c * _XR_PER_CHUNK, _XR_PER_CHUNK)],
                idx_v[b],
            )
            for j in range(_XR_PER_CHUNK):
                pltpu.async_copy(
                    table_hbm.at[idx_v[b].at[j]],
                    rows_v[b].at[j],
                    gsem[b],
                )

        def drain_gathers(b):
            for j in range(_XR_PER_CHUNK):
                pltpu.make_async_copy(
                    table_hbm.at[idx_v[b].at[j]],
                    rows_v[b].at[j],
                    gsem[b],
                ).wait()

        def scale_buf(b):
            @pl.loop(0, T)
            def scale_loop(t):
                for j in range(_XR_PER_CHUNK):
                    for l in range(D // 16):
                        rows_v[b][j, t, pl.ds(l * 16, 16)] = (
                            rows_v[b][j, t, pl.ds(l * 16, 16)] * scale
                        )

        def fire_writeback(c, b):
            pltpu.async_copy(
                rows_v[b],
                out_hbm.at[pl.ds(row0 + c * _XR_PER_CHUNK, _XR_PER_CHUNK)],
                wsem[b],
            )

        def wait_writeback(c, b):
            pltpu.make_async_copy(
                rows_v[b],
                out_hbm.at[pl.ds(row0 + c * _XR_PER_CHUNK, _XR_PER_CHUNK)],
                wsem[b],
            ).wait()

        stage_and_fire(0, 0)
        stage_and_fire(1, 1)

        @pl.loop(0, n_chunks - 2)
        def chunk_loop(c):
            b = lax.rem(c, 2)

            @pl.when(b == 0)
            def _():
                drain_gathers(0)
                scale_buf(0)
                fire_writeback(c, 0)
                wait_writeback(c, 0)
                stage_and_fire(c + 2, 0)

            @pl.when(b == 1)
            def _():
                drain_gathers(1)
                scale_buf(1)
                fire_writeback(c, 1)
                wait_writeback(c, 1)
                stage_and_fire(c + 2, 1)

        for c, b in ((n_chunks - 2, 0), (n_chunks - 1, 1)):
            drain_gathers(b)
            scale_buf(b)
            fire_writeback(c, b)
            wait_writeback(c, b)

    return lookup


def kernel(x, table):
    N, T = x.shape
    V, D = table.shape
    return _make_lookup(N, T, V, D)(x.astype(jnp.int32), table)
